# baseline (speedup 1.0000x reference)
import jax
import jax.numpy as jnp
from jax import lax
from jax.experimental import pallas as pl
from jax.experimental.pallas import tpu as pltpu
from jax.experimental.pallas import tpu_sc as plsc

_HARD_RATIO = 0.3
_MIN_KEPT = 1000
_FOCAL_ALPHA = 0.25
_DICE_WEIGHT = 0.5
_FOCAL_WEIGHT = 0.5

_B = 16
_NPIX = 512 * 512
_K = min(max(int(_NPIX * _HARD_RATIO), _MIN_KEPT), _NPIX)

_HALF = _NPIX // 2
_CH = 16384
_NCH = _HALF // _CH
_L1B = 2048
_L2B = 1024
_L3B = 1024


def _tc_body(pred_ref, target_ref, focal_ref, inter_ref, ssig_ref, st_ref):
    x = pred_ref[0]
    t = target_ref[0].astype(jnp.float32)

    bce = jnp.maximum(x, 0.0) - x * t + jnp.log1p(jnp.exp(-jnp.abs(x)))
    p_t = jnp.exp(-bce)
    focal_ref[0] = _FOCAL_ALPHA * (1.0 - p_t) ** 2 * bce

    sig = 1.0 / (1.0 + jnp.exp(-x))
    inter_ref[0, 0, 0] = jnp.sum(sig * t)
    ssig_ref[0, 0, 0] = jnp.sum(sig)
    st_ref[0, 0, 0] = jnp.sum(t)


def _sc_body(focal_hbm, out_hbm,
             dbuf, cnt_h, sum_h, comb_cnt, comb_sum, pair_cnt, pair_sum,
             ctrlbuf, resbuf, bs_smem, sh_cnt, sh_sum, sh_ctrl):
    c = lax.axis_index("c")
    s = lax.axis_index("s")
    sample = c * 8 + s // 2
    half = s % 2
    is_odd = half == 1
    is_even = half == 0
    base = sample * _NPIX + half * _HALF

    lane = lax.iota(jnp.int32, 16)
    ones_i = jnp.full((16,), 1, jnp.int32)
    zeros_i = jnp.zeros((16,), jnp.int32)
    zeros_f = jnp.zeros((16,), jnp.float32)

    def hist_pass(nb, key_fn, mask_fn):
        def zbody(z, _):
            cnt_h[pl.ds(z * 16, 16)] = zeros_i
            sum_h[pl.ds(z * 16, 16)] = zeros_f
            return 0
        lax.fori_loop(0, nb, zbody, 0)

        for ch in range(_NCH):
            pltpu.sync_copy(focal_hbm.at[pl.ds(base + ch * _CH, _CH)], dbuf)

            def ibody(i, _):
                v = dbuf[pl.ds(i * 16, 16)]
                bits = plsc.bitcast(v, jnp.int32)
                key = key_fn(bits)
                idx = key + lane * nb
                m = mask_fn(bits)
                if m is None:
                    plsc.addupdate_scatter(cnt_h, [idx], ones_i)
                    plsc.addupdate_scatter(sum_h, [idx], v)
                else:
                    plsc.addupdate_scatter(cnt_h, [idx], ones_i, mask=m)
                    plsc.addupdate_scatter(sum_h, [idx], v, mask=m)
                return 0
            lax.fori_loop(0, _CH // 16, ibody, 0)

        def cbody(j, _):
            acc_c = cnt_h[pl.ds(j * 16, 16)]
            acc_s = sum_h[pl.ds(j * 16, 16)]
            for l in range(1, 16):
                acc_c = acc_c + cnt_h[pl.ds(l * nb + j * 16, 16)]
                acc_s = acc_s + sum_h[pl.ds(l * nb + j * 16, 16)]
            comb_cnt[pl.ds(j * 16, 16)] = acc_c
            comb_sum[pl.ds(j * 16, 16)] = acc_s
            return 0
        lax.fori_loop(0, nb // 16, cbody, 0)

    def merge_pair(nb):
        @pl.when(is_odd)
        def _():
            pltpu.sync_copy(comb_cnt.at[pl.ds(0, nb)],
                            sh_cnt.at[sample, pl.ds(0, nb)])
            pltpu.sync_copy(comb_sum.at[pl.ds(0, nb)],
                            sh_sum.at[sample, pl.ds(0, nb)])
        plsc.subcore_barrier()

        @pl.when(is_even)
        def _():
            pltpu.sync_copy(sh_cnt.at[sample, pl.ds(0, nb)],
                            pair_cnt.at[pl.ds(0, nb)])
            pltpu.sync_copy(sh_sum.at[sample, pl.ds(0, nb)],
                            pair_sum.at[pl.ds(0, nb)])

            def abody(j, _):
                comb_cnt[pl.ds(j * 16, 16)] = (
                    comb_cnt[pl.ds(j * 16, 16)] + pair_cnt[pl.ds(j * 16, 16)])
                comb_sum[pl.ds(j * 16, 16)] = (
                    comb_sum[pl.ds(j * 16, 16)] + pair_sum[pl.ds(j * 16, 16)])
                return 0
            lax.fori_loop(0, nb // 16, abody, 0)

    def scan_level(nb, k_lvl):
        nblk = nb // 16

        def bsbody(j, _):
            bs_smem[j] = jnp.sum(comb_cnt[pl.ds(j * 16, 16)])
            return 0
        lax.fori_loop(0, nblk, bsbody, 0)

        def blkbody(jj, carry):
            suffix, blk, above = carry
            j = nblk - 1 - jj
            s_new = suffix + bs_smem[j]
            hit = jnp.logical_and(s_new >= k_lvl, suffix < k_lvl)
            blk = jnp.where(hit, j, blk)
            above = jnp.where(hit, suffix, above)
            return s_new, blk, above
        _, blk, above_blk = lax.fori_loop(
            0, nblk, blkbody, (jnp.int32(0), jnp.int32(0), jnp.int32(0)))
        blk = jnp.clip(blk, 0, nblk - 1)

        c16 = comb_cnt[pl.ds(blk * 16, 16)]
        r = lax.rev(c16, (0,))
        cs = plsc.cumsum(r)
        ge_count = jnp.full((16,), above_blk, jnp.int32) + cs
        m = ge_count >= jnp.full((16,), k_lvl, jnp.int32)
        p = plsc.all_reduce_population_count(m)
        loc = p - 1
        bstar_vec = jnp.full((16,), blk * 16, jnp.int32) + loc
        in_above = jnp.where(lane > loc, c16, 0)
        cnt_above = above_blk + jnp.sum(in_above)

        def sabody(j, acc):
            idxv = lane + j * 16
            msk = idxv > bstar_vec
            return acc + jnp.where(msk, comb_sum[pl.ds(j * 16, 16)], 0.0)
        acc = lax.fori_loop(0, nblk, sabody, zeros_f)
        sum_above = jnp.sum(acc)
        return bstar_vec, cnt_above, sum_above

    def bcast_ctrl(val):
        @pl.when(is_even)
        def _():
            ctrlbuf[...] = val
            pltpu.sync_copy(ctrlbuf, sh_ctrl.at[sample])
        plsc.subcore_barrier()
        pltpu.sync_copy(sh_ctrl.at[sample], ctrlbuf)
        return ctrlbuf[...]

    k_i = jnp.int32(_K)

    hist_pass(_L1B, lambda b: b >> 20, lambda b: None)
    merge_pair(_L1B)
    b1, cnt1, sum1 = scan_level(_L1B, k_i)
    b1v = bcast_ctrl(b1)

    k2 = k_i - cnt1
    hist_pass(_L2B, lambda b: (b >> 10) & 1023, lambda b: (b >> 20) == b1v)
    merge_pair(_L2B)
    b2, cnt2, sum2 = scan_level(_L2B, k2)
    b2v = bcast_ctrl(b2)
    p2v = b1v * 1024 + b2v

    k3 = k2 - cnt2
    hist_pass(_L3B, lambda b: b & 1023, lambda b: (b >> 10) == p2v)
    merge_pair(_L3B)
    b3, cnt3, sum3 = scan_level(_L3B, k3)

    @pl.when(is_even)
    def _():
        t_bits = b1 * (1 << 20) + b2 * (1 << 10) + b3
        tvec = plsc.bitcast(t_bits, jnp.float32)
        n_rem = (k3 - cnt3).astype(jnp.float32)
        resbuf[...] = (jnp.full((16,), sum1 + sum2 + sum3, jnp.float32)
                       + jnp.full((16,), n_rem, jnp.float32) * tvec)
        pltpu.sync_copy(resbuf, out_hbm.at[sample])


@jax.jit
def kernel(pred, target):
    pred2 = pred.reshape(_B, 2048, 128)
    target2 = target.reshape(_B, 2048, 128)

    scalar_spec = pl.BlockSpec((1, 1, 1), lambda i: (i, 0, 0),
                               memory_space=pltpu.SMEM)
    focal, inter, ssig, st = pl.pallas_call(
        _tc_body,
        grid=(_B,),
        in_specs=[
            pl.BlockSpec((1, 2048, 128), lambda i: (i, 0, 0)),
            pl.BlockSpec((1, 2048, 128), lambda i: (i, 0, 0)),
        ],
        out_specs=[pl.BlockSpec((1, 2048, 128), lambda i: (i, 0, 0)),
                   scalar_spec, scalar_spec, scalar_spec],
        out_shape=[jax.ShapeDtypeStruct((_B, 2048, 128), jnp.float32)]
        + [jax.ShapeDtypeStruct((_B, 1, 1), jnp.float32)] * 3,
    )(pred2, target2)

    sc_fn = pl.kernel(
        _sc_body,
        out_type=jax.ShapeDtypeStruct((_B, 16), jnp.float32),
        mesh=plsc.VectorSubcoreMesh(core_axis_name="c", subcore_axis_name="s",
                                    num_cores=2, num_subcores=16),
        compiler_params=pltpu.CompilerParams(needs_layout_passes=False),
        scratch_types=[
            pltpu.VMEM((_CH,), jnp.float32),
            pltpu.VMEM((_L1B * 16,), jnp.int32),
            pltpu.VMEM((_L1B * 16,), jnp.float32),
            pltpu.VMEM((_L1B,), jnp.int32),
            pltpu.VMEM((_L1B,), jnp.float32),
            pltpu.VMEM((_L1B,), jnp.int32),
            pltpu.VMEM((_L1B,), jnp.float32),
            pltpu.VMEM((16,), jnp.int32),
            pltpu.VMEM((16,), jnp.float32),
            pltpu.SMEM((_L1B // 16,), jnp.int32),
            pltpu.VMEM_SHARED((_B, _L1B), jnp.int32),
            pltpu.VMEM_SHARED((_B, _L1B), jnp.float32),
            pltpu.VMEM_SHARED((_B, 16), jnp.int32),
        ],
    )
    topk = sc_fn(focal.reshape(_B * _NPIX))[:, 0]

    hard_focal = jnp.sum(topk) / jnp.float32(_B * _K)
    dice = (2.0 * inter + 1.0) / (ssig + st + 1.0)
    dice_loss = jnp.mean(1.0 - dice)
    return _DICE_WEIGHT * dice_loss + _FOCAL_WEIGHT * hard_focal


# speedup vs baseline: 1.1479x; 1.1479x over previous
import jax
import jax.numpy as jnp
from jax import lax
from jax.experimental import pallas as pl
from jax.experimental.pallas import tpu as pltpu
from jax.experimental.pallas import tpu_sc as plsc

_FOCAL_ALPHA = 0.25
_DICE_WEIGHT = 0.5
_FOCAL_WEIGHT = 0.5

_B = 16
_NPIX = 512 * 512
_K = min(max(int(_NPIX * 0.3), 1000), _NPIX)

_HALF = _NPIX // 2
_CH = 16384
_NCH = _HALF // _CH
_L1B = 2048
_L2B = 1024
_L3B = 1024
_U = 8


def _tc_body(pred_ref, target_ref, focal_ref, inter_ref, ssig_ref, st_ref):
    x = pred_ref[0]
    t = target_ref[0].astype(jnp.float32)

    bce = jnp.maximum(x, 0.0) - x * t + jnp.log1p(jnp.exp(-jnp.abs(x)))
    p_t = jnp.exp(-bce)
    focal_ref[0] = _FOCAL_ALPHA * (1.0 - p_t) ** 2 * bce

    sig = 1.0 / (1.0 + jnp.exp(-x))
    inter_ref[0, 0, 0] = jnp.sum(sig * t)
    ssig_ref[0, 0, 0] = jnp.sum(sig)
    st_ref[0, 0, 0] = jnp.sum(t)


def _sc_body(focal_hbm, out_hbm,
             dbuf0, dbuf1, sem0, sem1, cnt_h, comb_cnt, pair_cnt, pair_sum,
             rbuf, bs_smem, sh_cnt, sh_sum):
    c = lax.axis_index("c")
    s = lax.axis_index("s")
    sample = c * 8 + s // 2
    half = s % 2
    is_odd = half == 1
    is_even = half == 0
    base = sample * _NPIX + half * _HALF

    lane = lax.iota(jnp.int32, 16)
    ones_i = jnp.full((16,), 1, jnp.int32)
    zeros_i = jnp.zeros((16,), jnp.int32)
    zeros_f = jnp.zeros((16,), jnp.float32)

    dbufs = (dbuf0, dbuf1)
    sems = (sem0, sem1)

    def stream(chunk_fn):
        cp = pltpu.async_copy(focal_hbm.at[pl.ds(base, _CH)], dbuf0, sem0)
        for ch in range(_NCH):
            if ch + 1 < _NCH:
                nxt = pltpu.async_copy(
                    focal_hbm.at[pl.ds(base + (ch + 1) * _CH, _CH)],
                    dbufs[(ch + 1) % 2], sems[(ch + 1) % 2])
            cp.wait()
            chunk_fn(dbufs[ch % 2])
            if ch + 1 < _NCH:
                cp = nxt

    def hist_pass(nb, key_fn, mask_fn):
        def zbody(z, _):
            for u in range(_U):
                cnt_h[pl.ds((z * _U + u) * 16, 16)] = zeros_i
            return 0
        lax.fori_loop(0, nb // _U, zbody, 0)

        laneoff = lane * nb

        def chunk_fn(dbuf):
            def ibody(i, _):
                for u in range(_U):
                    v = dbuf[pl.ds((i * _U + u) * 16, 16)]
                    bits = plsc.bitcast(v, jnp.int32)
                    idx = key_fn(bits) + laneoff
                    m = mask_fn(bits)
                    if m is None:
                        plsc.addupdate_scatter(cnt_h, [idx], ones_i)
                    else:
                        plsc.addupdate_scatter(cnt_h, [idx], ones_i, mask=m)
                return 0
            lax.fori_loop(0, _CH // 16 // _U, ibody, 0)
        stream(chunk_fn)

        def cbody(j, _):
            acc_c = cnt_h[pl.ds(j * 16, 16)]
            for l in range(1, 16):
                acc_c = acc_c + cnt_h[pl.ds(l * nb + j * 16, 16)]
            comb_cnt[pl.ds(j * 16, 16)] = acc_c
            return 0
        lax.fori_loop(0, nb // 16, cbody, 0)

    def merge_pair(nb):
        @pl.when(is_odd)
        def _():
            pltpu.sync_copy(comb_cnt, sh_cnt.at[sample])
        plsc.subcore_barrier()

        @pl.when(is_even)
        def _():
            pltpu.sync_copy(sh_cnt.at[sample], pair_cnt)

            def abody(j, _):
                comb_cnt[pl.ds(j * 16, 16)] = (
                    comb_cnt[pl.ds(j * 16, 16)] + pair_cnt[pl.ds(j * 16, 16)])
                return 0
            lax.fori_loop(0, nb // 16, abody, 0)

    def scan_level(nb, k_lvl):
        nblk = nb // 16

        def bsbody(j, _):
            bs_smem[j] = jnp.sum(comb_cnt[pl.ds(j * 16, 16)])
            return 0
        lax.fori_loop(0, nblk, bsbody, 0)

        def blkbody(jj, carry):
            suffix, blk, above = carry
            j = nblk - 1 - jj
            s_new = suffix + bs_smem[j]
            hit = jnp.logical_and(s_new >= k_lvl, suffix < k_lvl)
            blk = jnp.where(hit, j, blk)
            above = jnp.where(hit, suffix, above)
            return s_new, blk, above
        _, blk, above_blk = lax.fori_loop(
            0, nblk, blkbody, (jnp.int32(0), jnp.int32(0), jnp.int32(0)))
        blk = jnp.clip(blk, 0, nblk - 1)

        c16 = comb_cnt[pl.ds(blk * 16, 16)]
        r = lax.rev(c16, (0,))
        cs = plsc.cumsum(r)
        ge_count = jnp.full((16,), above_blk, jnp.int32) + cs
        m = ge_count >= jnp.full((16,), k_lvl, jnp.int32)
        p = plsc.all_reduce_population_count(m)
        loc = p - 1
        bstar_vec = jnp.full((16,), blk * 16, jnp.int32) + loc
        in_above = jnp.where(lane > loc, c16, 0)
        cnt_above = above_blk + jnp.sum(in_above)
        return bstar_vec, cnt_above

    def bcast_ctrl(val):
        @pl.when(is_even)
        def _():
            def fbody(j, _):
                pair_cnt[pl.ds(j * 16, 16)] = val
                return 0
            lax.fori_loop(0, _L1B // 16, fbody, 0)
            pltpu.sync_copy(pair_cnt, sh_cnt.at[sample])
        plsc.subcore_barrier()
        pltpu.sync_copy(sh_cnt.at[sample], pair_cnt)
        return pair_cnt[pl.ds(0, 16)]

    k_i = jnp.int32(_K)

    hist_pass(_L1B, lambda b: b >> 20, lambda b: None)
    merge_pair(_L1B)
    b1, cnt1 = scan_level(_L1B, k_i)
    b1v = bcast_ctrl(b1)

    k2 = k_i - cnt1
    hist_pass(_L2B, lambda b: (b >> 10) & 1023, lambda b: (b >> 20) == b1v)
    merge_pair(_L2B)
    b2, cnt2 = scan_level(_L2B, k2)
    b2v = bcast_ctrl(b2)
    p2v = b1v * 1024 + b2v

    k3 = k2 - cnt2
    hist_pass(_L3B, lambda b: b & 1023, lambda b: (b >> 10) == p2v)
    merge_pair(_L3B)
    b3, cnt3 = scan_level(_L3B, k3)
    b3v = bcast_ctrl(b3)

    tvec = plsc.bitcast(b1v * (1 << 20) + b2v * (1 << 10) + b3v, jnp.float32)

    carry0 = tuple([zeros_f] * _U) + tuple([zeros_f] * _U)

    def sum_chunk(dbuf, carry):
        def ibody(i, carry):
            accs = list(carry[:_U])
            accf = list(carry[_U:])
            for u in range(_U):
                v = dbuf[pl.ds((i * _U + u) * 16, 16)]
                m = v > tvec
                accs[u] = accs[u] + jnp.where(m, v, 0.0)
                accf[u] = accf[u] + jnp.where(m, 1.0, 0.0)
            return tuple(accs) + tuple(accf)
        return lax.fori_loop(0, _CH // 16 // _U, ibody, carry)

    carry = carry0
    cp = pltpu.async_copy(focal_hbm.at[pl.ds(base, _CH)], dbuf0, sem0)
    for ch in range(_NCH):
        if ch + 1 < _NCH:
            nxt = pltpu.async_copy(
                focal_hbm.at[pl.ds(base + (ch + 1) * _CH, _CH)],
                dbufs[(ch + 1) % 2], sems[(ch + 1) % 2])
        cp.wait()
        carry = sum_chunk(dbufs[ch % 2], carry)
        if ch + 1 < _NCH:
            cp = nxt

    accs = zeros_f
    accf = zeros_f
    for u in range(_U):
        accs = accs + carry[u]
        accf = accf + carry[_U + u]

    @pl.when(is_odd)
    def _():
        def fbody(j, _):
            pair_sum[pl.ds(j * 16, 16)] = jnp.where(
                jnp.full((16,), j, jnp.int32) == 0, accs, accf)
            return 0
        lax.fori_loop(0, _L1B // 16, fbody, 0)
        pltpu.sync_copy(pair_sum, sh_sum.at[sample])
    plsc.subcore_barrier()

    @pl.when(is_even)
    def _():
        pltpu.sync_copy(sh_sum.at[sample], pair_sum)
        psum = pair_sum[pl.ds(0, 16)]
        pcnt = pair_sum[pl.ds(16, 16)]
        sum_gt = jnp.sum(accs + psum)
        cnt_gt = jnp.sum(accf + pcnt)
        n_rem = jnp.float32(_K) - cnt_gt
        res = (jnp.full((16,), sum_gt, jnp.float32)
               + jnp.full((16,), n_rem, jnp.float32) * tvec)
        for j in range(4):
            rbuf[pl.ds(j * 16, 16)] = res
        pltpu.sync_copy(rbuf, out_hbm.at[sample])


@jax.jit
def kernel(pred, target):
    pred2 = pred.reshape(_B, 2048, 128)
    target2 = target.reshape(_B, 2048, 128)

    scalar_spec = pl.BlockSpec((1, 1, 1), lambda i: (i, 0, 0),
                               memory_space=pltpu.SMEM)
    focal, inter, ssig, st = pl.pallas_call(
        _tc_body,
        grid=(_B,),
        in_specs=[
            pl.BlockSpec((1, 2048, 128), lambda i: (i, 0, 0)),
            pl.BlockSpec((1, 2048, 128), lambda i: (i, 0, 0)),
        ],
        out_specs=[pl.BlockSpec((1, 2048, 128), lambda i: (i, 0, 0)),
                   scalar_spec, scalar_spec, scalar_spec],
        out_shape=[jax.ShapeDtypeStruct((_B, 2048, 128), jnp.float32)]
        + [jax.ShapeDtypeStruct((_B, 1, 1), jnp.float32)] * 3,
    )(pred2, target2)

    sc_fn = pl.kernel(
        _sc_body,
        out_type=jax.ShapeDtypeStruct((_B, 64), jnp.float32),
        mesh=plsc.VectorSubcoreMesh(core_axis_name="c", subcore_axis_name="s",
                                    num_cores=2, num_subcores=16),
        compiler_params=pltpu.CompilerParams(needs_layout_passes=False),
        scratch_types=[
            pltpu.VMEM((_CH,), jnp.float32),
            pltpu.VMEM((_CH,), jnp.float32),
            pltpu.SemaphoreType.DMA,
            pltpu.SemaphoreType.DMA,
            pltpu.VMEM((_L1B * 16,), jnp.int32),
            pltpu.VMEM((_L1B,), jnp.int32),
            pltpu.VMEM((_L1B,), jnp.int32),
            pltpu.VMEM((_L1B,), jnp.float32),
            pltpu.VMEM((64,), jnp.float32),
            pltpu.SMEM((_L1B // 16,), jnp.int32),
            pltpu.VMEM_SHARED((_B, _L1B), jnp.int32),
            pltpu.VMEM_SHARED((_B, _L1B), jnp.float32),
        ],
    )
    topk = sc_fn(focal.reshape(_B * _NPIX))[:, 0]

    hard_focal = jnp.sum(topk) / jnp.float32(_B * _K)
    dice = (2.0 * inter + 1.0) / (ssig + st + 1.0)
    dice_loss = jnp.mean(1.0 - dice)
    return _DICE_WEIGHT * dice_loss + _FOCAL_WEIGHT * hard_focal


# speedup vs baseline: 1.4993x; 1.3062x over previous
import jax
import jax.numpy as jnp
from jax import lax
from jax.experimental import pallas as pl
from jax.experimental.pallas import tpu as pltpu
from jax.experimental.pallas import tpu_sc as plsc

_FOCAL_ALPHA = 0.25
_DICE_WEIGHT = 0.5
_FOCAL_WEIGHT = 0.5

_B = 16
_NPIX = 512 * 512
_K = min(max(int(_NPIX * 0.3), 1000), _NPIX)

_HALF = _NPIX // 2
_CH = 16384
_NCH = _HALF // _CH
_L1B = 2048
_L2B = 1024
_L3B = 1024
_U = 8
_CAP = 2048


def _tc_body(pred_ref, target_ref, focal_ref, inter_ref, ssig_ref, st_ref):
    x = pred_ref[0]
    t = target_ref[0].astype(jnp.float32)

    bce = jnp.maximum(x, 0.0) - x * t + jnp.log1p(jnp.exp(-jnp.abs(x)))
    p_t = jnp.exp(-bce)
    focal_ref[0] = _FOCAL_ALPHA * (1.0 - p_t) ** 2 * bce

    sig = 1.0 / (1.0 + jnp.exp(-x))
    inter_ref[0, 0, 0] = jnp.sum(sig * t)
    ssig_ref[0, 0, 0] = jnp.sum(sig)
    st_ref[0, 0, 0] = jnp.sum(t)


def _sc_body(focal_hbm, out_hbm,
             dbuf0, dbuf1, sem0, sem1, cnt_h, cbuf, comb_cnt, pair_cnt,
             pair_sum, rbuf, bs_smem, sh_cnt, sh_sum):
    c = lax.axis_index("c")
    s = lax.axis_index("s")
    sample = c * 8 + s // 2
    half = s % 2
    is_odd = half == 1
    is_even = half == 0
    base = sample * _NPIX + half * _HALF

    lane = lax.iota(jnp.int32, 16)
    ones_i = jnp.full((16,), 1, jnp.int32)
    zeros_i = jnp.zeros((16,), jnp.int32)
    zeros_f = jnp.zeros((16,), jnp.float32)

    dbufs = (dbuf0, dbuf1)
    sems = (sem0, sem1)

    def stream(chunk_fn, carry):
        cp = pltpu.async_copy(focal_hbm.at[pl.ds(base, _CH)], dbuf0, sem0)
        for ch in range(_NCH):
            if ch + 1 < _NCH:
                nxt = pltpu.async_copy(
                    focal_hbm.at[pl.ds(base + (ch + 1) * _CH, _CH)],
                    dbufs[(ch + 1) % 2], sems[(ch + 1) % 2])
            cp.wait()
            carry = chunk_fn(dbufs[ch % 2], carry)
            if ch + 1 < _NCH:
                cp = nxt
        return carry

    def zero_hist(nb):
        def zbody(z, _):
            for u in range(_U):
                cnt_h[pl.ds((z * _U + u) * 16, 16)] = zeros_i
            return 0
        lax.fori_loop(0, nb // _U, zbody, 0)

    def combine_lanes(nb):
        def cbody(j, _):
            acc_c = cnt_h[pl.ds(j * 16, 16)]
            for l in range(1, 16):
                acc_c = acc_c + cnt_h[pl.ds(l * nb + j * 16, 16)]
            comb_cnt[pl.ds(j * 16, 16)] = acc_c
            return 0
        lax.fori_loop(0, nb // 16, cbody, 0)

    def merge_pair(nb):
        @pl.when(is_odd)
        def _():
            pltpu.sync_copy(comb_cnt, sh_cnt.at[sample])
        plsc.subcore_barrier()

        @pl.when(is_even)
        def _():
            pltpu.sync_copy(sh_cnt.at[sample], pair_cnt)

            def abody(j, _):
                comb_cnt[pl.ds(j * 16, 16)] = (
                    comb_cnt[pl.ds(j * 16, 16)] + pair_cnt[pl.ds(j * 16, 16)])
                return 0
            lax.fori_loop(0, nb // 16, abody, 0)

    def scan_level(nb, k_lvl):
        nblk = nb // 16

        def bsbody(j, _):
            bs_smem[j] = jnp.sum(comb_cnt[pl.ds(j * 16, 16)])
            return 0
        lax.fori_loop(0, nblk, bsbody, 0)

        def blkbody(jj, carry):
            suffix, blk, above = carry
            j = nblk - 1 - jj
            s_new = suffix + bs_smem[j]
            hit = jnp.logical_and(s_new >= k_lvl, suffix < k_lvl)
            blk = jnp.where(hit, j, blk)
            above = jnp.where(hit, suffix, above)
            return s_new, blk, above
        _, blk, above_blk = lax.fori_loop(
            0, nblk, blkbody, (jnp.int32(0), jnp.int32(0), jnp.int32(0)))
        blk = jnp.clip(blk, 0, nblk - 1)

        c16 = comb_cnt[pl.ds(blk * 16, 16)]
        r = lax.rev(c16, (0,))
        cs = plsc.cumsum(r)
        ge_count = jnp.full((16,), above_blk, jnp.int32) + cs
        m = ge_count >= jnp.full((16,), k_lvl, jnp.int32)
        p = plsc.all_reduce_population_count(m)
        loc = p - 1
        bstar_vec = jnp.full((16,), blk * 16, jnp.int32) + loc
        in_above = jnp.where(lane > loc, c16, 0)
        cnt_above = above_blk + jnp.sum(in_above)
        return bstar_vec, cnt_above

    def bcast_ctrl(val):
        @pl.when(is_even)
        def _():
            def fbody(j, _):
                pair_cnt[pl.ds(j * 16, 16)] = val
                return 0
            lax.fori_loop(0, _L1B // 16, fbody, 0)
            pltpu.sync_copy(pair_cnt, sh_cnt.at[sample])
        plsc.subcore_barrier()
        pltpu.sync_copy(sh_cnt.at[sample], pair_cnt)
        return pair_cnt[pl.ds(0, 16)]

    k_i = jnp.int32(_K)

    zero_hist(_L1B)
    laneoff = lane * _L1B

    def chunkA(dbuf, carry):
        def ibody(i, _):
            for u in range(_U):
                v = dbuf[pl.ds((i * _U + u) * 16, 16)]
                bits = plsc.bitcast(v, jnp.int32)
                plsc.addupdate_scatter(cnt_h, [(bits >> 20) + laneoff], ones_i)
            return 0
        lax.fori_loop(0, _CH // 16 // _U, ibody, 0)
        return carry
    stream(chunkA, 0)
    combine_lanes(_L1B)
    merge_pair(_L1B)
    b1, cnt1 = scan_level(_L1B, k_i)
    b1v = bcast_ctrl(b1)

    def chunkB(dbuf, carry):
        def ibody(i, carry):
            s_hi = list(carry[:_U])
            ofs = carry[_U]
            for u in range(_U):
                v = dbuf[pl.ds((i * _U + u) * 16, 16)]
                bits = plsc.bitcast(v, jnp.int32)
                key = bits >> 20
                m_gt = key > b1v
                m_eq = key == b1v
                s_hi[u] = s_hi[u] + jnp.where(m_gt, v, 0.0)
                idxo = jnp.minimum(ofs, _CAP - 1) * 16 + lane
                plsc.store_scatter(cbuf, [idxo], v, mask=m_eq)
                ofs = ofs + jnp.where(m_eq, 1, 0)
            return tuple(s_hi) + (ofs,)
        return lax.fori_loop(0, _CH // 16 // _U, ibody, carry)

    carry = stream(chunkB, tuple([zeros_f] * _U) + (zeros_i,))
    s_hi = zeros_f
    for u in range(_U):
        s_hi = s_hi + carry[u]
    ofs = carry[_U]
    jmax = jnp.max(jnp.minimum(ofs, _CAP))

    zero_hist(_L2B)
    laneoff2 = lane * _L2B

    def cbodyC(j, _):
        v = cbuf[pl.ds(j * 16, 16)]
        bits = plsc.bitcast(v, jnp.int32)
        valid = ofs > jnp.full((16,), j, jnp.int32)
        plsc.addupdate_scatter(cnt_h, [((bits >> 10) & 1023) + laneoff2],
                               ones_i, mask=valid)
        return 0
    lax.fori_loop(0, jmax, cbodyC, 0)
    combine_lanes(_L2B)
    merge_pair(_L2B)
    k2 = k_i - cnt1
    b2, cnt2 = scan_level(_L2B, k2)
    b2v = bcast_ctrl(b2)

    zero_hist(_L3B)
    laneoff3 = lane * _L3B

    def cbodyD(j, _):
        v = cbuf[pl.ds(j * 16, 16)]
        bits = plsc.bitcast(v, jnp.int32)
        valid = ofs > jnp.full((16,), j, jnp.int32)
        m = jnp.logical_and(valid, ((bits >> 10) & 1023) == b2v)
        plsc.addupdate_scatter(cnt_h, [(bits & 1023) + laneoff3],
                               ones_i, mask=m)
        return 0
    lax.fori_loop(0, jmax, cbodyD, 0)
    combine_lanes(_L3B)
    merge_pair(_L3B)
    k3 = k2 - cnt2
    b3, cnt3 = scan_level(_L3B, k3)
    b3v = bcast_ctrl(b3)

    tvec = plsc.bitcast(b1v * (1 << 20) + b2v * (1 << 10) + b3v, jnp.float32)

    def cbodyE(j, carry):
        s_in, c_in = carry
        v = cbuf[pl.ds(j * 16, 16)]
        valid = ofs > jnp.full((16,), j, jnp.int32)
        m = jnp.logical_and(valid, v > tvec)
        return s_in + jnp.where(m, v, 0.0), c_in + jnp.where(m, 1.0, 0.0)
    s_in, c_in = lax.fori_loop(0, jmax, cbodyE, (zeros_f, zeros_f))

    s_part = s_hi + s_in

    @pl.when(is_odd)
    def _():
        def fbody(j, _):
            pair_sum[pl.ds(j * 16, 16)] = jnp.where(
                jnp.full((16,), j, jnp.int32) == 0, s_part, c_in)
            return 0
        lax.fori_loop(0, _L1B // 16, fbody, 0)
        pltpu.sync_copy(pair_sum, sh_sum.at[sample])
    plsc.subcore_barrier()

    @pl.when(is_even)
    def _():
        pltpu.sync_copy(sh_sum.at[sample], pair_sum)
        psum = pair_sum[pl.ds(0, 16)]
        pcnt = pair_sum[pl.ds(16, 16)]
        sum_gt = jnp.sum(s_part + psum)
        cnt_gt = cnt1.astype(jnp.float32) + jnp.sum(c_in + pcnt)
        n_rem = jnp.float32(_K) - cnt_gt
        res = (jnp.full((16,), sum_gt, jnp.float32)
               + jnp.full((16,), n_rem, jnp.float32) * tvec)
        for j in range(4):
            rbuf[pl.ds(j * 16, 16)] = res
        pltpu.sync_copy(rbuf, out_hbm.at[sample])


@jax.jit
def kernel(pred, target):
    pred2 = pred.reshape(_B, 2048, 128)
    target2 = target.reshape(_B, 2048, 128)

    scalar_spec = pl.BlockSpec((1, 1, 1), lambda i: (i, 0, 0),
                               memory_space=pltpu.SMEM)
    focal, inter, ssig, st = pl.pallas_call(
        _tc_body,
        grid=(_B,),
        in_specs=[
            pl.BlockSpec((1, 2048, 128), lambda i: (i, 0, 0)),
            pl.BlockSpec((1, 2048, 128), lambda i: (i, 0, 0)),
        ],
        out_specs=[pl.BlockSpec((1, 2048, 128), lambda i: (i, 0, 0)),
                   scalar_spec, scalar_spec, scalar_spec],
        out_shape=[jax.ShapeDtypeStruct((_B, 2048, 128), jnp.float32)]
        + [jax.ShapeDtypeStruct((_B, 1, 1), jnp.float32)] * 3,
    )(pred2, target2)

    sc_fn = pl.kernel(
        _sc_body,
        out_type=jax.ShapeDtypeStruct((_B, 64), jnp.float32),
        mesh=plsc.VectorSubcoreMesh(core_axis_name="c", subcore_axis_name="s",
                                    num_cores=2, num_subcores=16),
        compiler_params=pltpu.CompilerParams(needs_layout_passes=False),
        scratch_types=[
            pltpu.VMEM((_CH,), jnp.float32),
            pltpu.VMEM((_CH,), jnp.float32),
            pltpu.SemaphoreType.DMA,
            pltpu.SemaphoreType.DMA,
            pltpu.VMEM((_L1B * 16,), jnp.int32),
            pltpu.VMEM((_CAP * 16,), jnp.float32),
            pltpu.VMEM((_L1B,), jnp.int32),
            pltpu.VMEM((_L1B,), jnp.int32),
            pltpu.VMEM((_L1B,), jnp.float32),
            pltpu.VMEM((64,), jnp.float32),
            pltpu.SMEM((_L1B // 16,), jnp.int32),
            pltpu.VMEM_SHARED((_B, _L1B), jnp.int32),
            pltpu.VMEM_SHARED((_B, _L1B), jnp.float32),
        ],
    )
    topk = sc_fn(focal.reshape(_B * _NPIX))[:, 0]

    hard_focal = jnp.sum(topk) / jnp.float32(_B * _K)
    dice = (2.0 * inter + 1.0) / (ssig + st + 1.0)
    dice_loss = jnp.mean(1.0 - dice)
    return _DICE_WEIGHT * dice_loss + _FOCAL_WEIGHT * hard_focal


# speedup vs baseline: 1.5199x; 1.0137x over previous
import jax
import jax.numpy as jnp
from jax import lax
from jax.experimental import pallas as pl
from jax.experimental.pallas import tpu as pltpu
from jax.experimental.pallas import tpu_sc as plsc

_FOCAL_ALPHA = 0.25
_DICE_WEIGHT = 0.5
_FOCAL_WEIGHT = 0.5

_B = 16
_NPIX = 512 * 512
_K = min(max(int(_NPIX * 0.3), 1000), _NPIX)

_HALF = _NPIX // 2
_CH = 16384
_NCH = _HALF // _CH
_L1B = 2048
_L2B = 1024
_L3B = 1024
_U = 8
_CAP = 2048


def _tc_body(pred_ref, target_ref, focal_ref, inter_ref, ssig_ref, st_ref):
    x = pred_ref[0]
    t = target_ref[0].astype(jnp.float32)

    bce = jnp.maximum(x, 0.0) - x * t + jnp.log1p(jnp.exp(-jnp.abs(x)))
    p_t = jnp.exp(-bce)
    focal_ref[0] = _FOCAL_ALPHA * (1.0 - p_t) ** 2 * bce

    sig = 1.0 / (1.0 + jnp.exp(-x))
    inter_ref[0, 0, 0] = jnp.sum(sig * t)
    ssig_ref[0, 0, 0] = jnp.sum(sig)
    st_ref[0, 0, 0] = jnp.sum(t)


def _sc_body(focal_hbm, out_hbm,
             dbuf0, dbuf1, sem0, sem1, cnt_h, cbuf, comb_cnt, pair_cnt,
             pair_sum, rbuf, bs_smem, sh_cnt, sh_sum):
    c = lax.axis_index("c")
    s = lax.axis_index("s")
    sample = c * 8 + s // 2
    half = s % 2
    is_odd = half == 1
    is_even = half == 0
    base = sample * _NPIX + half * _HALF

    lane = lax.iota(jnp.int32, 16)
    ones_i = jnp.full((16,), 1, jnp.int32)
    zeros_i = jnp.zeros((16,), jnp.int32)
    zeros_f = jnp.zeros((16,), jnp.float32)

    dbufs = (dbuf0, dbuf1)
    sems = (sem0, sem1)

    def stream(chunk_fn, carry):
        cp = pltpu.async_copy(focal_hbm.at[pl.ds(base, _CH)], dbuf0, sem0)
        for ch in range(_NCH):
            if ch + 1 < _NCH:
                nxt = pltpu.async_copy(
                    focal_hbm.at[pl.ds(base + (ch + 1) * _CH, _CH)],
                    dbufs[(ch + 1) % 2], sems[(ch + 1) % 2])
            cp.wait()
            carry = chunk_fn(dbufs[ch % 2], carry)
            if ch + 1 < _NCH:
                cp = nxt
        return carry

    def zero_hist(nb):
        def zbody(z, _):
            for u in range(_U):
                cnt_h[pl.ds((z * _U + u) * 16, 16)] = zeros_i
            return 0
        lax.fori_loop(0, nb // _U, zbody, 0)

    def combine_lanes(nb):
        def cbody(j, _):
            acc_c = cnt_h[pl.ds(j * 16, 16)]
            for l in range(1, 16):
                acc_c = acc_c + cnt_h[pl.ds(l * nb + j * 16, 16)]
            comb_cnt[pl.ds(j * 16, 16)] = acc_c
            return 0
        lax.fori_loop(0, nb // 16, cbody, 0)

    def merge_pair(nb):
        @pl.when(is_odd)
        def _():
            pltpu.sync_copy(comb_cnt, sh_cnt.at[sample])
        plsc.subcore_barrier()

        @pl.when(is_even)
        def _():
            pltpu.sync_copy(sh_cnt.at[sample], pair_cnt)

            def abody(j, _):
                comb_cnt[pl.ds(j * 16, 16)] = (
                    comb_cnt[pl.ds(j * 16, 16)] + pair_cnt[pl.ds(j * 16, 16)])
                return 0
            lax.fori_loop(0, nb // 16, abody, 0)

    def scan_level(nb, k_lvl):
        nblk = nb // 16

        def bsbody(j, _):
            bs_smem[j] = jnp.sum(comb_cnt[pl.ds(j * 16, 16)])
            return 0
        lax.fori_loop(0, nblk, bsbody, 0)

        def blkbody(jj, carry):
            suffix, blk, above = carry
            j = nblk - 1 - jj
            s_new = suffix + bs_smem[j]
            hit = jnp.logical_and(s_new >= k_lvl, suffix < k_lvl)
            blk = jnp.where(hit, j, blk)
            above = jnp.where(hit, suffix, above)
            return s_new, blk, above
        _, blk, above_blk = lax.fori_loop(
            0, nblk, blkbody, (jnp.int32(0), jnp.int32(0), jnp.int32(0)))
        blk = jnp.clip(blk, 0, nblk - 1)

        c16 = comb_cnt[pl.ds(blk * 16, 16)]
        r = lax.rev(c16, (0,))
        cs = plsc.cumsum(r)
        ge_count = jnp.full((16,), above_blk, jnp.int32) + cs
        m = ge_count >= jnp.full((16,), k_lvl, jnp.int32)
        p = plsc.all_reduce_population_count(m)
        loc = p - 1
        bstar_vec = jnp.full((16,), blk * 16, jnp.int32) + loc
        in_above = jnp.where(lane > loc, c16, 0)
        cnt_above = above_blk + jnp.sum(in_above)
        return bstar_vec, cnt_above

    def bcast_ctrl(val):
        @pl.when(is_even)
        def _():
            pair_cnt[pl.ds(0, 16)] = val
            pltpu.sync_copy(pair_cnt, sh_cnt.at[sample])
        plsc.subcore_barrier()
        pltpu.sync_copy(sh_cnt.at[sample], pair_cnt)
        return pair_cnt[pl.ds(0, 16)]

    k_i = jnp.int32(_K)

    zero_hist(_L1B)
    laneoff = lane * _L1B

    def chunkA(dbuf, carry):
        def ibody(i, _):
            for u in range(_U):
                v = dbuf[pl.ds((i * _U + u) * 16, 16)]
                bits = plsc.bitcast(v, jnp.int32)
                plsc.addupdate_scatter(cnt_h, [(bits >> 20) + laneoff], ones_i)
            return 0
        lax.fori_loop(0, _CH // 16 // _U, ibody, 0)
        return carry
    stream(chunkA, 0)
    combine_lanes(_L1B)
    merge_pair(_L1B)
    b1, cnt1 = scan_level(_L1B, k_i)
    b1v = bcast_ctrl(b1)

    def chunkB(dbuf, carry):
        def ibody(i, carry):
            s_hi = list(carry[:_U])
            ofs16 = carry[_U]
            for u in range(_U):
                v = dbuf[pl.ds((i * _U + u) * 16, 16)]
                bits = plsc.bitcast(v, jnp.int32)
                key = bits >> 20
                m_gt = key > b1v
                m_eq = key == b1v
                s_hi[u] = s_hi[u] + jnp.where(m_gt, v, 0.0)
                idxo = jnp.minimum(ofs16, (_CAP - 1) * 16) + lane
                plsc.store_scatter(cbuf, [idxo], v, mask=m_eq)
                ofs16 = ofs16 + jnp.where(m_eq, 16, 0)
            return tuple(s_hi) + (ofs16,)
        return lax.fori_loop(0, _CH // 16 // _U, ibody, carry)

    carry = stream(chunkB, tuple([zeros_f] * _U) + (zeros_i,))
    s_hi = zeros_f
    for u in range(_U):
        s_hi = s_hi + carry[u]
    ofs = carry[_U] >> 4
    jmax = jnp.max(jnp.minimum(ofs, _CAP))

    zero_hist(_L2B)
    laneoff2 = lane * _L2B

    def cbodyC(j, _):
        v = cbuf[pl.ds(j * 16, 16)]
        bits = plsc.bitcast(v, jnp.int32)
        valid = ofs > jnp.full((16,), j, jnp.int32)
        plsc.addupdate_scatter(cnt_h, [((bits >> 10) & 1023) + laneoff2],
                               ones_i, mask=valid)
        return 0
    lax.fori_loop(0, jmax, cbodyC, 0)
    combine_lanes(_L2B)
    merge_pair(_L2B)
    k2 = k_i - cnt1
    b2, cnt2 = scan_level(_L2B, k2)
    b2v = bcast_ctrl(b2)

    zero_hist(_L3B)
    laneoff3 = lane * _L3B

    def cbodyD(j, _):
        v = cbuf[pl.ds(j * 16, 16)]
        bits = plsc.bitcast(v, jnp.int32)
        valid = ofs > jnp.full((16,), j, jnp.int32)
        m = jnp.logical_and(valid, ((bits >> 10) & 1023) == b2v)
        plsc.addupdate_scatter(cnt_h, [(bits & 1023) + laneoff3],
                               ones_i, mask=m)
        return 0
    lax.fori_loop(0, jmax, cbodyD, 0)
    combine_lanes(_L3B)
    merge_pair(_L3B)
    k3 = k2 - cnt2
    b3, cnt3 = scan_level(_L3B, k3)
    b3v = bcast_ctrl(b3)

    tvec = plsc.bitcast(b1v * (1 << 20) + b2v * (1 << 10) + b3v, jnp.float32)

    def cbodyE(j, carry):
        s_in, c_in = carry
        v = cbuf[pl.ds(j * 16, 16)]
        valid = ofs > jnp.full((16,), j, jnp.int32)
        m = jnp.logical_and(valid, v > tvec)
        return s_in + jnp.where(m, v, 0.0), c_in + jnp.where(m, 1.0, 0.0)
    s_in, c_in = lax.fori_loop(0, jmax, cbodyE, (zeros_f, zeros_f))

    s_part = s_hi + s_in

    @pl.when(is_odd)
    def _():
        pair_sum[pl.ds(0, 16)] = s_part
        pair_sum[pl.ds(16, 16)] = c_in
        pltpu.sync_copy(pair_sum, sh_sum.at[sample])
    plsc.subcore_barrier()

    @pl.when(is_even)
    def _():
        pltpu.sync_copy(sh_sum.at[sample], pair_sum)
        psum = pair_sum[pl.ds(0, 16)]
        pcnt = pair_sum[pl.ds(16, 16)]
        sum_gt = jnp.sum(s_part + psum)
        cnt_gt = cnt1.astype(jnp.float32) + jnp.sum(c_in + pcnt)
        n_rem = jnp.float32(_K) - cnt_gt
        res = (jnp.full((16,), sum_gt, jnp.float32)
               + jnp.full((16,), n_rem, jnp.float32) * tvec)
        for j in range(4):
            rbuf[pl.ds(j * 16, 16)] = res
        pltpu.sync_copy(rbuf, out_hbm.at[sample])


@jax.jit
def kernel(pred, target):
    pred2 = pred.reshape(_B, 2048, 128)
    target2 = target.reshape(_B, 2048, 128)

    scalar_spec = pl.BlockSpec((1, 1, 1), lambda i: (i, 0, 0),
                               memory_space=pltpu.SMEM)
    focal, inter, ssig, st = pl.pallas_call(
        _tc_body,
        grid=(_B,),
        in_specs=[
            pl.BlockSpec((1, 2048, 128), lambda i: (i, 0, 0)),
            pl.BlockSpec((1, 2048, 128), lambda i: (i, 0, 0)),
        ],
        out_specs=[pl.BlockSpec((1, 2048, 128), lambda i: (i, 0, 0)),
                   scalar_spec, scalar_spec, scalar_spec],
        out_shape=[jax.ShapeDtypeStruct((_B, 2048, 128), jnp.float32)]
        + [jax.ShapeDtypeStruct((_B, 1, 1), jnp.float32)] * 3,
    )(pred2, target2)

    sc_fn = pl.kernel(
        _sc_body,
        out_type=jax.ShapeDtypeStruct((_B, 64), jnp.float32),
        mesh=plsc.VectorSubcoreMesh(core_axis_name="c", subcore_axis_name="s",
                                    num_cores=2, num_subcores=16),
        compiler_params=pltpu.CompilerParams(needs_layout_passes=False),
        scratch_types=[
            pltpu.VMEM((_CH,), jnp.float32),
            pltpu.VMEM((_CH,), jnp.float32),
            pltpu.SemaphoreType.DMA,
            pltpu.SemaphoreType.DMA,
            pltpu.VMEM((_L1B * 16,), jnp.int32),
            pltpu.VMEM((_CAP * 16,), jnp.float32),
            pltpu.VMEM((_L1B,), jnp.int32),
            pltpu.VMEM((_L1B,), jnp.int32),
            pltpu.VMEM((_L1B,), jnp.float32),
            pltpu.VMEM((64,), jnp.float32),
            pltpu.SMEM((_L1B // 16,), jnp.int32),
            pltpu.VMEM_SHARED((_B, _L1B), jnp.int32),
            pltpu.VMEM_SHARED((_B, _L1B), jnp.float32),
        ],
    )
    topk = sc_fn(focal.reshape(_B * _NPIX))[:, 0]

    hard_focal = jnp.sum(topk) / jnp.float32(_B * _K)
    dice = (2.0 * inter + 1.0) / (ssig + st + 1.0)
    dice_loss = jnp.mean(1.0 - dice)
    return _DICE_WEIGHT * dice_loss + _FOCAL_WEIGHT * hard_focal
